# baseline (speedup 1.0000x reference)
import functools

import jax
import jax.numpy as jnp
from jax import lax
from jax.experimental import pallas as pl
from jax.experimental.pallas import tpu as pltpu
from jax.experimental.pallas import tpu_sc as plsc

N_FIELDS = 26
B = 16384
VOCAB = 100000
D = 16
IN_DIM = 10
HID = 128

NC = 2
NS = 16
NW = NC * NS
BPW = B // NW
CHUNK = 128
NCHUNK = BPW // CHUNK


def _sc_body(zoff_hbm, wm_hbm, wlv_hbm, eps_hbm, out_hbm,
             idx_v, mean_v, lv_v, eps_v, buf_v, sem_g, sem_io):
    wid = lax.axis_index("s") * NC + lax.axis_index("c")
    base = wid * BPW

    for s in range(NCHUNK):
        row0 = base + s * CHUNK
        pltpu.sync_copy(zoff_hbm.at[wid, s], idx_v)

        def field(f, carry):
            eps_cp = pltpu.async_copy(
                eps_hbm.at[f, pl.ds(row0, CHUNK)], eps_v, sem_io)
            m_cp = pltpu.async_copy(wm_hbm.at[idx_v.at[f]], mean_v, sem_g)
            l_cp = pltpu.async_copy(wlv_hbm.at[idx_v.at[f]], lv_v, sem_g)
            m_cp.wait()
            l_cp.wait()
            eps_cp.wait()

            def row(i, c):
                m = mean_v[i, :]
                l = lv_v[i, :]
                e = eps_v[i, :]
                buf_v[i, pl.ds(f * D, D)] = m + jnp.exp(l * 0.5) * e
                return c

            lax.fori_loop(0, CHUNK, row, 0, unroll=4)
            return carry

        lax.fori_loop(0, N_FIELDS, field, 0)
        pltpu.sync_copy(buf_v, out_hbm.at[pl.ds(row0, CHUNK), :])


_sc_gather = functools.partial(
    pl.kernel,
    out_type=jax.ShapeDtypeStruct((B, N_FIELDS * D), jnp.float32),
    mesh=plsc.VectorSubcoreMesh(core_axis_name="c", subcore_axis_name="s"),
    compiler_params=pltpu.CompilerParams(use_tc_tiling_on_sc=False),
    scratch_types=[
        pltpu.VMEM((N_FIELDS, CHUNK), jnp.int32),
        pltpu.VMEM((CHUNK, D), jnp.float32),
        pltpu.VMEM((CHUNK, D), jnp.float32),
        pltpu.VMEM((CHUNK, D), jnp.float32),
        pltpu.VMEM((CHUNK, N_FIELDS * D), jnp.float32),
        pltpu.SemaphoreType.DMA,
        pltpu.SemaphoreType.DMA,
    ],
)(_sc_body)


BLK = 2048


def _mlp_body(x_ref, z_ref, w1x_ref, w1z_ref, b1_ref, w2_ref, b2_ref, o_ref):
    h = jnp.dot(x_ref[...], w1x_ref[...], preferred_element_type=jnp.float32)
    h = h + jnp.dot(z_ref[...], w1z_ref[...],
                    preferred_element_type=jnp.float32)
    h = jnp.maximum(h + b1_ref[...], 0.0)
    o_ref[...] = jnp.sum(h * w2_ref[...], axis=1, keepdims=True) + b2_ref[...]


def kernel(X, y, Z, eps, W_mean, W_log_var, W1, b1, W2, b2):
    offs = (jnp.arange(N_FIELDS, dtype=jnp.int32) * VOCAB)[:, None]
    zoff = (Z + offs).reshape(N_FIELDS, NW, NCHUNK, CHUNK).transpose(1, 2, 0, 3)
    wm = W_mean.reshape(N_FIELDS * VOCAB, D)
    wlv = W_log_var.reshape(N_FIELDS * VOCAB, D)

    zcat = _sc_gather(zoff, wm, wlv, eps)

    w1x = W1[:IN_DIM]
    w1z = W1[IN_DIM:]
    out = pl.pallas_call(
        _mlp_body,
        grid=(B // BLK,),
        in_specs=[
            pl.BlockSpec((BLK, IN_DIM), lambda i: (i, 0)),
            pl.BlockSpec((BLK, N_FIELDS * D), lambda i: (i, 0)),
            pl.BlockSpec((IN_DIM, HID), lambda i: (0, 0)),
            pl.BlockSpec((N_FIELDS * D, HID), lambda i: (0, 0)),
            pl.BlockSpec((1, HID), lambda i: (0, 0)),
            pl.BlockSpec((1, HID), lambda i: (0, 0)),
            pl.BlockSpec((1, 1), lambda i: (0, 0)),
        ],
        out_specs=pl.BlockSpec((BLK, 1), lambda i: (i, 0)),
        out_shape=jax.ShapeDtypeStruct((B, 1), jnp.float32),
    )(X, zcat, w1x, w1z, b1.reshape(1, HID), W2.reshape(1, HID),
      b2.reshape(1, 1))
    return out


# speedup vs baseline: 1.1106x; 1.1106x over previous
import functools

import jax
import jax.numpy as jnp
from jax import lax
from jax.experimental import pallas as pl
from jax.experimental.pallas import tpu as pltpu
from jax.experimental.pallas import tpu_sc as plsc

N_FIELDS = 26
B = 16384
VOCAB = 100000
D = 16
IN_DIM = 10
HID = 128
FD = N_FIELDS * D

NC = 2
NS = 16
NW = NC * NS
BPW = B // NW
CHUNK = 128
NCHUNK = BPW // CHUNK


def _sc_body(zoff_hbm, wm_hbm, wlv_hbm, out_hbm, idx_v, gbuf, sem_g, sem_w):
    wid = lax.axis_index("s") * NC + lax.axis_index("c")
    base = wid * BPW

    for s in range(NCHUNK):
        row0 = base + s * CHUNK
        pltpu.sync_copy(zoff_hbm.at[wid, s], idx_v)

        def fire(f, carry):
            pltpu.async_copy(
                wm_hbm.at[idx_v.at[f]],
                gbuf.at[pl.ds(2 * f * CHUNK, CHUNK)], sem_g)
            pltpu.async_copy(
                wlv_hbm.at[idx_v.at[f]],
                gbuf.at[pl.ds((2 * f + 1) * CHUNK, CHUNK)], sem_g)
            return carry

        lax.fori_loop(0, N_FIELDS, fire, 0)
        pltpu.make_async_copy(
            wm_hbm.at[pl.ds(0, 2 * N_FIELDS * CHUNK)], gbuf, sem_g).wait()

        def wback(f, carry):
            pltpu.async_copy(
                gbuf.at[pl.ds(2 * f * CHUNK, CHUNK)],
                out_hbm.at[pl.ds(row0, CHUNK), pl.ds(f * D, D)], sem_w)
            pltpu.async_copy(
                gbuf.at[pl.ds((2 * f + 1) * CHUNK, CHUNK)],
                out_hbm.at[pl.ds(row0, CHUNK), pl.ds(FD + f * D, D)], sem_w)
            return carry

        lax.fori_loop(0, N_FIELDS, wback, 0)
        pltpu.make_async_copy(
            wm_hbm.at[pl.ds(0, 2 * N_FIELDS * CHUNK)], gbuf, sem_w).wait()


_sc_gather = functools.partial(
    pl.kernel,
    out_type=jax.ShapeDtypeStruct((B, 2 * FD), jnp.float32),
    mesh=plsc.VectorSubcoreMesh(core_axis_name="c", subcore_axis_name="s"),
    compiler_params=pltpu.CompilerParams(use_tc_tiling_on_sc=False),
    scratch_types=[
        pltpu.VMEM((N_FIELDS, CHUNK), jnp.int32),
        pltpu.VMEM((2 * N_FIELDS * CHUNK, D), jnp.float32),
        pltpu.SemaphoreType.DMA,
        pltpu.SemaphoreType.DMA,
    ],
)(_sc_body)


BLK = 2048


def _mlp_body(x_ref, g_ref, e_ref, w1x_ref, w1z_ref, b1_ref, w2_ref, b2_ref,
              o_ref):
    z = g_ref[:, :FD] + jnp.exp(g_ref[:, FD:] * 0.5) * e_ref[...]
    h = jnp.dot(x_ref[...], w1x_ref[...], preferred_element_type=jnp.float32)
    h = h + jnp.dot(z, w1z_ref[...], preferred_element_type=jnp.float32)
    h = jnp.maximum(h + b1_ref[...], 0.0)
    o_ref[...] = jnp.sum(h * w2_ref[...], axis=1, keepdims=True) + b2_ref[...]


def kernel(X, y, Z, eps, W_mean, W_log_var, W1, b1, W2, b2):
    offs = (jnp.arange(N_FIELDS, dtype=jnp.int32) * VOCAB)[:, None]
    zoff = (Z + offs).reshape(N_FIELDS, NW, NCHUNK, CHUNK).transpose(1, 2, 0, 3)
    wm = W_mean.reshape(N_FIELDS * VOCAB, D)
    wlv = W_log_var.reshape(N_FIELDS * VOCAB, D)

    gathered = _sc_gather(zoff, wm, wlv)
    eps_cat = eps.transpose(1, 0, 2).reshape(B, FD)

    w1x = W1[:IN_DIM]
    w1z = W1[IN_DIM:]
    out = pl.pallas_call(
        _mlp_body,
        grid=(B // BLK,),
        in_specs=[
            pl.BlockSpec((BLK, IN_DIM), lambda i: (i, 0)),
            pl.BlockSpec((BLK, 2 * FD), lambda i: (i, 0)),
            pl.BlockSpec((BLK, FD), lambda i: (i, 0)),
            pl.BlockSpec((IN_DIM, HID), lambda i: (0, 0)),
            pl.BlockSpec((FD, HID), lambda i: (0, 0)),
            pl.BlockSpec((1, HID), lambda i: (0, 0)),
            pl.BlockSpec((1, HID), lambda i: (0, 0)),
            pl.BlockSpec((1, 1), lambda i: (0, 0)),
        ],
        out_specs=pl.BlockSpec((BLK, 1), lambda i: (i, 0)),
        out_shape=jax.ShapeDtypeStruct((B, 1), jnp.float32),
    )(X, gathered, eps_cat, w1x, w1z, b1.reshape(1, HID), W2.reshape(1, HID),
      b2.reshape(1, 1))
    return out


# speedup vs baseline: 2.2739x; 2.0475x over previous
import functools

import jax
import jax.numpy as jnp
from jax import lax
from jax.experimental import pallas as pl
from jax.experimental.pallas import tpu as pltpu
from jax.experimental.pallas import tpu_sc as plsc

N_FIELDS = 26
B = 16384
VOCAB = 100000
D = 16
IN_DIM = 10
HID = 128
FD = N_FIELDS * D

NC = 2
NS = 16
NW = NC * NS
CH = 4096
NCH = B // CH
VL = 16


def _sc_body(z_hbm, wmT_hbm, wlvT_hbm, out_hbm, plane_v, idx_v, val_v):
    wid = lax.axis_index("s") * NC + lax.axis_index("c")
    d = wid % NS

    def run(tbl_hbm):
        def field(f, carry):
            pltpu.sync_copy(tbl_hbm.at[f, d], plane_v)

            def chunk(c, carry2):
                pltpu.sync_copy(z_hbm.at[f, pl.ds(c * CH, CH)], idx_v)

                def vec(i, carry3):
                    idx = idx_v[pl.ds(i * VL, VL)]
                    val_v[pl.ds(i * VL, VL)] = plsc.load_gather(plane_v, [idx])
                    return carry3

                lax.fori_loop(0, CH // VL, vec, 0, unroll=8)
                pltpu.sync_copy(val_v, out_hbm.at[f, wid, pl.ds(c * CH, CH)])
                return carry2

            lax.fori_loop(0, NCH, chunk, 0)
            return carry

        lax.fori_loop(0, N_FIELDS, field, 0)

    @pl.when(wid < NS)
    def _():
        run(wmT_hbm)

    @pl.when(wid >= NS)
    def _():
        run(wlvT_hbm)


_sc_gather = functools.partial(
    pl.kernel,
    out_type=jax.ShapeDtypeStruct((N_FIELDS, NW, B), jnp.float32),
    mesh=plsc.VectorSubcoreMesh(core_axis_name="c", subcore_axis_name="s"),
    compiler_params=pltpu.CompilerParams(
        use_tc_tiling_on_sc=False, needs_layout_passes=False),
    scratch_types=[
        pltpu.VMEM((VOCAB,), jnp.float32),
        pltpu.VMEM((CH,), jnp.int32),
        pltpu.VMEM((CH,), jnp.float32),
    ],
)(_sc_body)


BLK = 2048


def _mlp_body(g_ref, e_ref, xt_ref, w1x_ref, w1z_ref, b1_ref, w2_ref, b2_ref,
              o_ref):
    m = g_ref[:, :NS, :].reshape(FD, BLK)
    l = g_ref[:, NS:, :].reshape(FD, BLK)
    e = e_ref[...].reshape(FD, BLK)
    z = m + jnp.exp(l * 0.5) * e
    h = lax.dot_general(w1z_ref[...], z, (((0,), (0,)), ((), ())),
                        preferred_element_type=jnp.float32)
    h = h + lax.dot_general(w1x_ref[...], xt_ref[...], (((0,), (0,)), ((), ())),
                            preferred_element_type=jnp.float32)
    h = jnp.maximum(h + b1_ref[...], 0.0)
    o_ref[...] = (jnp.sum(h * w2_ref[...], axis=0, keepdims=True)
                  + b2_ref[...])


def kernel(X, y, Z, eps, W_mean, W_log_var, W1, b1, W2, b2):
    wmT = W_mean.transpose(0, 2, 1)
    wlvT = W_log_var.transpose(0, 2, 1)
    gathered = _sc_gather(Z, wmT, wlvT)

    epsT = eps.transpose(0, 2, 1)
    xT = X.transpose(1, 0)

    w1x = W1[:IN_DIM]
    w1z = W1[IN_DIM:]
    out = pl.pallas_call(
        _mlp_body,
        grid=(B // BLK,),
        in_specs=[
            pl.BlockSpec((N_FIELDS, NW, BLK), lambda i: (0, 0, i)),
            pl.BlockSpec((N_FIELDS, D, BLK), lambda i: (0, 0, i)),
            pl.BlockSpec((IN_DIM, BLK), lambda i: (0, i)),
            pl.BlockSpec((IN_DIM, HID), lambda i: (0, 0)),
            pl.BlockSpec((FD, HID), lambda i: (0, 0)),
            pl.BlockSpec((HID, 1), lambda i: (0, 0)),
            pl.BlockSpec((HID, 1), lambda i: (0, 0)),
            pl.BlockSpec((1, 1), lambda i: (0, 0)),
        ],
        out_specs=pl.BlockSpec((1, BLK), lambda i: (0, i)),
        out_shape=jax.ShapeDtypeStruct((1, B), jnp.float32),
    )(gathered, epsT, xT, w1x, w1z, b1.reshape(HID, 1), W2, b2.reshape(1, 1))
    return out.reshape(B, 1)


# speedup vs baseline: 2.9733x; 1.3076x over previous
import functools

import jax
import jax.numpy as jnp
from jax import lax
from jax.experimental import pallas as pl
from jax.experimental.pallas import tpu as pltpu
from jax.experimental.pallas import tpu_sc as plsc

N_FIELDS = 26
B = 16384
VOCAB = 100000
D = 16
IN_DIM = 10
HID = 128
FD = N_FIELDS * D

NC = 2
NS = 16
NW = NC * NS
FPT = 13
CH = 4096
NCH = B // CH
VL = 16


def _sc_body(z_hbm, tbl_hbm, out_hbm, plane_v, idx_v, obuf_v, sem_p, sem_o):
    wid = lax.axis_index("s") * NC + lax.axis_index("c")
    d = wid % NS
    f0 = (wid // NS) * FPT

    def field(j, carry):
        f = f0 + j
        cp_p = pltpu.async_copy(tbl_hbm.at[f, d], plane_v, sem_p)
        cp_i = pltpu.async_copy(z_hbm.at[f], idx_v, sem_p)
        cp_p.wait()
        cp_i.wait()

        def chunk(c, carry2):
            @pl.when(jnp.logical_or(c >= 2, j > 0))
            def _():
                pltpu.make_async_copy(
                    obuf_v.at[c % 2], out_hbm.at[0, 0, pl.ds(0, CH)],
                    sem_o).wait()

            def vec(i, carry3):
                idx = idx_v[pl.ds(c * CH + i * VL, VL)]
                obuf_v[c % 2, pl.ds(i * VL, VL)] = plsc.load_gather(
                    plane_v, [idx])
                return carry3

            lax.fori_loop(0, CH // VL, vec, 0, unroll=8)
            pltpu.async_copy(
                obuf_v.at[c % 2], out_hbm.at[f, d, pl.ds(c * CH, CH)], sem_o)
            return carry2

        lax.fori_loop(0, NCH, chunk, 0)
        return carry

    lax.fori_loop(0, FPT, field, 0)
    pltpu.make_async_copy(
        obuf_v.at[0], out_hbm.at[0, 0, pl.ds(0, CH)], sem_o).wait()
    pltpu.make_async_copy(
        obuf_v.at[1], out_hbm.at[0, 0, pl.ds(0, CH)], sem_o).wait()


_sc_gather = functools.partial(
    pl.kernel,
    out_type=jax.ShapeDtypeStruct((N_FIELDS, NS, B), jnp.float32),
    mesh=plsc.VectorSubcoreMesh(core_axis_name="c", subcore_axis_name="s"),
    compiler_params=pltpu.CompilerParams(
        use_tc_tiling_on_sc=False, needs_layout_passes=False),
    scratch_types=[
        pltpu.VMEM((VOCAB,), jnp.float32),
        pltpu.VMEM((B,), jnp.int32),
        pltpu.VMEM((2, CH), jnp.float32),
        pltpu.SemaphoreType.DMA,
        pltpu.SemaphoreType.DMA,
    ],
)(_sc_body)


BLK = 2048


def _mlp_body(mg_ref, lg_ref, e_ref, xt_ref, w1x_ref, w1z_ref, b1_ref,
              w2_ref, b2_ref, o_ref):
    m = mg_ref[...].reshape(FD, BLK)
    l = lg_ref[...].reshape(FD, BLK)
    e = e_ref[...].reshape(FD, BLK)
    z = m + jnp.exp(l * 0.5) * e
    h = lax.dot_general(w1z_ref[...], z, (((0,), (0,)), ((), ())),
                        preferred_element_type=jnp.float32)
    h = h + lax.dot_general(w1x_ref[...], xt_ref[...], (((0,), (0,)), ((), ())),
                            preferred_element_type=jnp.float32)
    h = jnp.maximum(h + b1_ref[...], 0.0)
    o_ref[...] = (jnp.sum(h * w2_ref[...], axis=0, keepdims=True)
                  + b2_ref[...])


def kernel(X, y, Z, eps, W_mean, W_log_var, W1, b1, W2, b2):
    wmT = W_mean.transpose(0, 2, 1)
    wlvT = W_log_var.transpose(0, 2, 1)
    mg = _sc_gather(Z, wmT)
    lg = _sc_gather(Z, wlvT)

    epsT = eps.transpose(0, 2, 1)
    xT = X.transpose(1, 0)

    w1x = W1[:IN_DIM]
    w1z = W1[IN_DIM:]
    out = pl.pallas_call(
        _mlp_body,
        grid=(B // BLK,),
        in_specs=[
            pl.BlockSpec((N_FIELDS, D, BLK), lambda i: (0, 0, i)),
            pl.BlockSpec((N_FIELDS, D, BLK), lambda i: (0, 0, i)),
            pl.BlockSpec((N_FIELDS, D, BLK), lambda i: (0, 0, i)),
            pl.BlockSpec((IN_DIM, BLK), lambda i: (0, i)),
            pl.BlockSpec((IN_DIM, HID), lambda i: (0, 0)),
            pl.BlockSpec((FD, HID), lambda i: (0, 0)),
            pl.BlockSpec((HID, 1), lambda i: (0, 0)),
            pl.BlockSpec((HID, 1), lambda i: (0, 0)),
            pl.BlockSpec((1, 1), lambda i: (0, 0)),
        ],
        out_specs=pl.BlockSpec((1, BLK), lambda i: (0, i)),
        out_shape=jax.ShapeDtypeStruct((1, B), jnp.float32),
    )(mg, lg, epsT, xT, w1x, w1z, b1.reshape(HID, 1), W2, b2.reshape(1, 1))
    return out.reshape(B, 1)


# speedup vs baseline: 2.9791x; 1.0019x over previous
import functools

import jax
import jax.numpy as jnp
from jax import lax
from jax.experimental import pallas as pl
from jax.experimental.pallas import tpu as pltpu
from jax.experimental.pallas import tpu_sc as plsc

N_FIELDS = 26
B = 16384
VOCAB = 100000
D = 16
IN_DIM = 10
HID = 128
FD = N_FIELDS * D

NC = 2
NS = 16
NW = NC * NS
FPT = 13
CH = 4096
NCH = B // CH
VL = 16


def _sc_body(z_hbm, tbl_hbm, out_hbm, plane_v, idx_v, obuf_v, sem_p, sem_o):
    wid = lax.axis_index("s") * NC + lax.axis_index("c")
    d = wid % NS
    f0 = (wid // NS) * FPT

    def field(j, carry):
        f = f0 + j
        cp_p = pltpu.async_copy(tbl_hbm.at[f, d], plane_v, sem_p)
        cp_i = pltpu.async_copy(z_hbm.at[f], idx_v, sem_p)
        cp_p.wait()
        cp_i.wait()

        def chunk(c, carry2):
            @pl.when(jnp.logical_or(c >= 2, j > 0))
            def _():
                pltpu.make_async_copy(
                    obuf_v.at[c % 2], out_hbm.at[0, 0, pl.ds(0, CH)],
                    sem_o).wait()

            def vec(i, carry3):
                idx = idx_v[pl.ds(c * CH + i * VL, VL)]
                obuf_v[c % 2, pl.ds(i * VL, VL)] = plsc.load_gather(
                    plane_v, [idx])
                return carry3

            lax.fori_loop(0, CH // VL, vec, 0, unroll=16)
            pltpu.async_copy(
                obuf_v.at[c % 2], out_hbm.at[f, d, pl.ds(c * CH, CH)], sem_o)
            return carry2

        lax.fori_loop(0, NCH, chunk, 0)
        return carry

    lax.fori_loop(0, FPT, field, 0)
    pltpu.make_async_copy(
        obuf_v.at[0], out_hbm.at[0, 0, pl.ds(0, CH)], sem_o).wait()
    pltpu.make_async_copy(
        obuf_v.at[1], out_hbm.at[0, 0, pl.ds(0, CH)], sem_o).wait()


_sc_gather = functools.partial(
    pl.kernel,
    out_type=jax.ShapeDtypeStruct((N_FIELDS, NS, B), jnp.float32),
    mesh=plsc.VectorSubcoreMesh(core_axis_name="c", subcore_axis_name="s"),
    compiler_params=pltpu.CompilerParams(
        use_tc_tiling_on_sc=False, needs_layout_passes=False,
        disable_bounds_checks=True),
    scratch_types=[
        pltpu.VMEM((VOCAB,), jnp.float32),
        pltpu.VMEM((B,), jnp.int32),
        pltpu.VMEM((2, CH), jnp.float32),
        pltpu.SemaphoreType.DMA,
        pltpu.SemaphoreType.DMA,
    ],
)(_sc_body)


BLK = 2048


def _mlp_body(mg_ref, lg_ref, e_ref, xt_ref, w1x_ref, w1z_ref, b1_ref,
              w2_ref, b2_ref, o_ref):
    m = mg_ref[...].reshape(FD, BLK)
    l = lg_ref[...].reshape(FD, BLK)
    e = e_ref[...].reshape(FD, BLK)
    z = m + jnp.exp(l * 0.5) * e
    h = lax.dot_general(w1z_ref[...], z, (((0,), (0,)), ((), ())),
                        preferred_element_type=jnp.float32)
    h = h + lax.dot_general(w1x_ref[...], xt_ref[...], (((0,), (0,)), ((), ())),
                            preferred_element_type=jnp.float32)
    h = jnp.maximum(h + b1_ref[...], 0.0)
    o_ref[...] = (jnp.sum(h * w2_ref[...], axis=0, keepdims=True)
                  + b2_ref[...])


def kernel(X, y, Z, eps, W_mean, W_log_var, W1, b1, W2, b2):
    wmT = W_mean.transpose(0, 2, 1)
    wlvT = W_log_var.transpose(0, 2, 1)
    mg = _sc_gather(Z, wmT)
    lg = _sc_gather(Z, wlvT)

    epsT = eps.transpose(0, 2, 1)
    xT = X.transpose(1, 0)

    w1x = W1[:IN_DIM]
    w1z = W1[IN_DIM:]
    out = pl.pallas_call(
        _mlp_body,
        grid=(B // BLK,),
        in_specs=[
            pl.BlockSpec((N_FIELDS, D, BLK), lambda i: (0, 0, i)),
            pl.BlockSpec((N_FIELDS, D, BLK), lambda i: (0, 0, i)),
            pl.BlockSpec((N_FIELDS, D, BLK), lambda i: (0, 0, i)),
            pl.BlockSpec((IN_DIM, BLK), lambda i: (0, i)),
            pl.BlockSpec((IN_DIM, HID), lambda i: (0, 0)),
            pl.BlockSpec((FD, HID), lambda i: (0, 0)),
            pl.BlockSpec((HID, 1), lambda i: (0, 0)),
            pl.BlockSpec((HID, 1), lambda i: (0, 0)),
            pl.BlockSpec((1, 1), lambda i: (0, 0)),
        ],
        out_specs=pl.BlockSpec((1, BLK), lambda i: (0, i)),
        out_shape=jax.ShapeDtypeStruct((1, B), jnp.float32),
    )(mg, lg, epsT, xT, w1x, w1z, b1.reshape(HID, 1), W2, b2.reshape(1, 1))
    return out.reshape(B, 1)


# speedup vs baseline: 5.2017x; 1.7461x over previous
import functools

import jax
import jax.numpy as jnp
from jax import lax
from jax.experimental import pallas as pl
from jax.experimental.pallas import tpu as pltpu
from jax.experimental.pallas import tpu_sc as plsc

N_FIELDS = 26
B = 16384
VOCAB = 100000
D = 16
IN_DIM = 10
HID = 128
FD = N_FIELDS * D

NC = 2
NS = 16
NW = NC * NS
FPT = 13
CH = 4096
NCH = B // CH
VL = 16


def _sc_body(z_hbm, tbl_hbm, out_hbm, plane_v, idx_v, obuf_v, sem_p, sem_o):
    wid = lax.axis_index("s") * NC + lax.axis_index("c")
    d = wid % NS
    f0 = (wid // NS) * FPT

    def field(j, carry):
        f = f0 + j
        cp_p = pltpu.async_copy(tbl_hbm.at[f, d], plane_v, sem_p)
        cp_i = pltpu.async_copy(z_hbm.at[f], idx_v, sem_p)
        cp_p.wait()
        cp_i.wait()

        def chunk(c, carry2):
            @pl.when(jnp.logical_or(c >= 2, j > 0))
            def _():
                pltpu.make_async_copy(
                    obuf_v.at[c % 2], out_hbm.at[0, 0, pl.ds(0, CH)],
                    sem_o).wait()

            def vec(i, carry3):
                idx = idx_v[pl.ds(c * CH + i * VL, VL)]
                obuf_v[c % 2, pl.ds(i * VL, VL)] = plsc.load_gather(
                    plane_v, [idx])
                return carry3

            lax.fori_loop(0, CH // VL, vec, 0, unroll=16)
            pltpu.async_copy(
                obuf_v.at[c % 2], out_hbm.at[f, d, pl.ds(c * CH, CH)], sem_o)
            return carry2

        lax.fori_loop(0, NCH, chunk, 0)
        return carry

    lax.fori_loop(0, FPT, field, 0)
    pltpu.make_async_copy(
        obuf_v.at[0], out_hbm.at[0, 0, pl.ds(0, CH)], sem_o).wait()
    pltpu.make_async_copy(
        obuf_v.at[1], out_hbm.at[0, 0, pl.ds(0, CH)], sem_o).wait()


_sc_gather = functools.partial(
    pl.kernel,
    out_type=jax.ShapeDtypeStruct((N_FIELDS, NS, B), jnp.float32),
    mesh=plsc.VectorSubcoreMesh(core_axis_name="c", subcore_axis_name="s"),
    compiler_params=pltpu.CompilerParams(
        use_tc_tiling_on_sc=True, needs_layout_passes=False,
        disable_bounds_checks=True),
    scratch_types=[
        pltpu.VMEM((VOCAB,), jnp.float32),
        pltpu.VMEM((B,), jnp.int32),
        pltpu.VMEM((2, CH), jnp.float32),
        pltpu.SemaphoreType.DMA,
        pltpu.SemaphoreType.DMA,
    ],
)(_sc_body)


BLK = 2048


def _mlp_body(mg_ref, lg_ref, e_ref, xt_ref, w1x_ref, w1z_ref, b1_ref,
              w2_ref, b2_ref, o_ref):
    m = mg_ref[...].reshape(FD, BLK)
    l = lg_ref[...].reshape(FD, BLK)
    e = e_ref[...].reshape(FD, BLK)
    z = m + jnp.exp(l * 0.5) * e
    h = lax.dot_general(w1z_ref[...], z, (((0,), (0,)), ((), ())),
                        preferred_element_type=jnp.float32)
    h = h + lax.dot_general(w1x_ref[...], xt_ref[...], (((0,), (0,)), ((), ())),
                            preferred_element_type=jnp.float32)
    h = jnp.maximum(h + b1_ref[...], 0.0)
    o_ref[...] = (jnp.sum(h * w2_ref[...], axis=0, keepdims=True)
                  + b2_ref[...])


def kernel(X, y, Z, eps, W_mean, W_log_var, W1, b1, W2, b2):
    wmT = W_mean.transpose(0, 2, 1)
    wlvT = W_log_var.transpose(0, 2, 1)
    mg = _sc_gather(Z, wmT)
    lg = _sc_gather(Z, wlvT)

    epsT = eps.transpose(0, 2, 1)
    xT = X.transpose(1, 0)

    w1x = W1[:IN_DIM]
    w1z = W1[IN_DIM:]
    out = pl.pallas_call(
        _mlp_body,
        grid=(B // BLK,),
        in_specs=[
            pl.BlockSpec((N_FIELDS, D, BLK), lambda i: (0, 0, i)),
            pl.BlockSpec((N_FIELDS, D, BLK), lambda i: (0, 0, i)),
            pl.BlockSpec((N_FIELDS, D, BLK), lambda i: (0, 0, i)),
            pl.BlockSpec((IN_DIM, BLK), lambda i: (0, i)),
            pl.BlockSpec((IN_DIM, HID), lambda i: (0, 0)),
            pl.BlockSpec((FD, HID), lambda i: (0, 0)),
            pl.BlockSpec((HID, 1), lambda i: (0, 0)),
            pl.BlockSpec((HID, 1), lambda i: (0, 0)),
            pl.BlockSpec((1, 1), lambda i: (0, 0)),
        ],
        out_specs=pl.BlockSpec((1, BLK), lambda i: (0, i)),
        out_shape=jax.ShapeDtypeStruct((1, B), jnp.float32),
    )(mg, lg, epsT, xT, w1x, w1z, b1.reshape(HID, 1), W2, b2.reshape(1, 1))
    return out.reshape(B, 1)
